# baseline (speedup 1.0000x reference)
import functools

import jax
import jax.numpy as jnp
from jax import lax
from jax.experimental import pallas as pl
from jax.experimental.pallas import tpu as pltpu
from jax.experimental.pallas import tpu_sc as plsc

EMB = 64
HID = 128
B = 1024
T = 50
NTOK = B * T
NPAIR = 500000
CHUNK = 80
LANES = 16


def _make_gather():
    info = plsc.get_sparse_core_info()
    nc, ns = info.num_cores, info.num_subcores
    nw = nc * ns
    tok_w = NTOK // nw
    nchunk = tok_w // CHUNK

    mesh = plsc.VectorSubcoreMesh(core_axis_name="c", subcore_axis_name="s")

    @functools.partial(
        pl.kernel,
        mesh=mesh,
        compiler_params=pltpu.CompilerParams(needs_layout_passes=False),
        out_type=jax.ShapeDtypeStruct((NTOK, 2 * EMB), jnp.float32),
        scratch_types=[
            pltpu.VMEM((tok_w,), jnp.int32),
            pltpu.VMEM((CHUNK,), jnp.int32),
            pltpu.VMEM((CHUNK, 2 * EMB), jnp.float32),
            pltpu.SemaphoreType.DMA,
        ],
    )
    def gather_k(table_hbm, idx_hbm, out_hbm, idx_v, pair_v, rows_v, sem):
        wid = lax.axis_index("s") * nc + lax.axis_index("c")
        base = wid * tok_w
        pltpu.sync_copy(idx_hbm.at[wid], idx_v)

        def do_chunk(g, carry):
            def one(q, c):
                iv = idx_v[pl.ds(g * CHUNK + q * LANES, LANES)]
                pair_v[pl.ds(q * LANES, LANES)] = (
                    lax.shift_right_logical(iv, 1))
                return c
            lax.fori_loop(0, CHUNK // LANES, one, 0, unroll=True)
            pltpu.async_copy(table_hbm.at[pair_v], rows_v, sem).wait()
            pltpu.sync_copy(rows_v,
                            out_hbm.at[pl.ds(base + g * CHUNK, CHUNK)])
            return carry
        lax.fori_loop(0, nchunk, do_chunk, 0)

    return gather_k


_gather = _make_gather()


def _sigmoid(x):
    return 0.5 * jnp.tanh(0.5 * x) + 0.5


def _lstm_body(x_ref, m_ref, wih_ref, whh_ref, bih_ref, bhh_ref, wcls_ref,
               bcls_ref, out_ref):
    wih = wih_ref[...].astype(jnp.bfloat16)
    whh = whh_ref[...].astype(jnp.bfloat16)
    b = bih_ref[...] + bhh_ref[...]

    def step(t, carry):
        h, c = carry
        xp = x_ref[t]
        m = m_ref[t]
        left = xp[:, :EMB]
        right = xp[:, EMB:]
        xt = (left + m * (right - left)).astype(jnp.bfloat16)
        gates = jnp.dot(xt, wih, preferred_element_type=jnp.float32)
        gates = gates + jnp.dot(h.astype(jnp.bfloat16), whh,
                                preferred_element_type=jnp.float32)
        gates = gates + b
        i = _sigmoid(gates[:, :HID])
        f = _sigmoid(gates[:, HID:2 * HID])
        g = jnp.tanh(gates[:, 2 * HID:3 * HID])
        o = _sigmoid(gates[:, 3 * HID:])
        c = f * c + i * g
        h = o * jnp.tanh(c)
        return (h, c)

    h0 = jnp.zeros((B, HID), jnp.float32)
    c0 = jnp.zeros((B, HID), jnp.float32)
    h, _ = lax.fori_loop(0, T, step, (h0, c0))
    out_ref[...] = (jnp.dot(h, wcls_ref[...], preferred_element_type=jnp.float32)
                    + bcls_ref[...])


def kernel(batch_input_ids, emb, W_ih, W_hh, b_ih, b_hh, W_cls, b_cls):
    idx_tb = batch_input_ids.T
    idx = idx_tb.reshape(32, NTOK // 32)
    one = b_cls[0] * jnp.float32(0.0) + jnp.float32(1.0)
    table2 = emb.reshape(NPAIR, 2 * EMB) * one
    pairs = _gather(table2, idx)
    x = pairs.reshape(T, B, 2 * EMB)
    m = lax.bitwise_and(idx_tb, 1).astype(jnp.float32).reshape(T, B, 1)

    nlbl = W_cls.shape[0]
    wcls_pad = jnp.zeros((HID, 128), jnp.float32).at[:, :nlbl].set(W_cls.T)
    bcls_pad = jnp.zeros((1, 128), jnp.float32).at[0, :nlbl].set(b_cls)

    out = pl.pallas_call(
        _lstm_body,
        out_shape=jax.ShapeDtypeStruct((B, 128), jnp.float32),
    )(x, m, W_ih.T, W_hh.T, b_ih.reshape(1, -1), b_hh.reshape(1, -1),
      wcls_pad, bcls_pad)
    return out[:, :nlbl]


# speedup vs baseline: 2.0251x; 2.0251x over previous
import functools

import jax
import jax.numpy as jnp
from jax import lax
from jax.experimental import pallas as pl
from jax.experimental.pallas import tpu as pltpu
from jax.experimental.pallas import tpu_sc as plsc

EMB = 64
HID = 128
B = 1024
T = 50
TH = T // 2
NTOKH = B * TH
CHUNK = 80
LANES = 16


def _make_gather():
    info = plsc.get_sparse_core_info()
    nc, ns = info.num_cores, info.num_subcores
    nw = nc * ns
    tok_w = NTOKH // nw
    nchunk = tok_w // CHUNK

    mesh = plsc.VectorSubcoreMesh(core_axis_name="c", subcore_axis_name="s")

    @functools.partial(
        pl.kernel,
        mesh=mesh,
        compiler_params=pltpu.CompilerParams(needs_layout_passes=False),
        out_type=jax.ShapeDtypeStruct((NTOKH, EMB), jnp.float32),
        scratch_types=[
            pltpu.VMEM((tok_w,), jnp.int32),
            pltpu.VMEM((CHUNK, EMB), jnp.float32),
            pltpu.SemaphoreType.DMA,
        ],
    )
    def gather_k(table_hbm, idx_hbm, out_hbm, idx_v, rows_v, sem):
        wid = lax.axis_index("s") * nc + lax.axis_index("c")
        base = wid * tok_w
        pltpu.sync_copy(idx_hbm.at[wid], idx_v)

        def do_chunk(g, carry):
            def fire(q, c):
                iv = idx_v[pl.ds(g * CHUNK + q * LANES, LANES)]
                for jj in range(LANES):
                    pltpu.async_copy(table_hbm.at[iv[jj]],
                                     rows_v.at[q * LANES + jj], sem)
                return c
            lax.fori_loop(0, CHUNK // LANES, fire, 0)

            def drain(j, c):
                pltpu.make_async_copy(table_hbm.at[0], rows_v.at[j],
                                      sem).wait()
                return c
            lax.fori_loop(0, CHUNK, drain, 0)
            pltpu.sync_copy(rows_v,
                            out_hbm.at[pl.ds(base + g * CHUNK, CHUNK)])
            return carry
        lax.fori_loop(0, nchunk, do_chunk, 0)

    return gather_k


_gather = _make_gather()


def _sigmoid(x):
    return 0.5 * jnp.tanh(0.5 * x) + 0.5


def _lstm_steps(x_ref, wih, whh, b, h, c):
    def step(t, carry):
        h, c = carry
        xt = x_ref[t].astype(jnp.bfloat16)
        gates = jnp.dot(xt, wih, preferred_element_type=jnp.float32)
        gates = gates + jnp.dot(h.astype(jnp.bfloat16), whh,
                                preferred_element_type=jnp.float32)
        gates = gates + b
        i = _sigmoid(gates[:, :HID])
        f = _sigmoid(gates[:, HID:2 * HID])
        g = jnp.tanh(gates[:, 2 * HID:3 * HID])
        o = _sigmoid(gates[:, 3 * HID:])
        c = f * c + i * g
        h = o * jnp.tanh(c)
        return (h, c)
    return lax.fori_loop(0, TH, step, (h, c))


def _lstm_first(x_ref, wih_ref, whh_ref, bih_ref, bhh_ref, h_out, c_out):
    wih = wih_ref[...].astype(jnp.bfloat16)
    whh = whh_ref[...].astype(jnp.bfloat16)
    b = bih_ref[...] + bhh_ref[...]
    h0 = jnp.zeros((B, HID), jnp.float32)
    c0 = jnp.zeros((B, HID), jnp.float32)
    h, c = _lstm_steps(x_ref, wih, whh, b, h0, c0)
    h_out[...] = h
    c_out[...] = c


def _lstm_second(x_ref, h_ref, c_ref, wih_ref, whh_ref, bih_ref, bhh_ref,
                 wcls_ref, bcls_ref, out_ref):
    wih = wih_ref[...].astype(jnp.bfloat16)
    whh = whh_ref[...].astype(jnp.bfloat16)
    b = bih_ref[...] + bhh_ref[...]
    h, c = _lstm_steps(x_ref, wih, whh, b, h_ref[...], c_ref[...])
    out_ref[...] = (jnp.dot(h, wcls_ref[...], preferred_element_type=jnp.float32)
                    + bcls_ref[...])


def kernel(batch_input_ids, emb, W_ih, W_hh, b_ih, b_hh, W_cls, b_cls):
    idx_tb = batch_input_ids.T
    idx_a = idx_tb[:TH].reshape(32, NTOKH // 32)
    idx_b = idx_tb[TH:].reshape(32, NTOKH // 32)
    xa = _gather(emb, idx_a).reshape(TH, B, EMB)
    xb = _gather(emb, idx_b).reshape(TH, B, EMB)

    nlbl = W_cls.shape[0]
    wcls_pad = jnp.zeros((HID, 128), jnp.float32).at[:, :nlbl].set(W_cls.T)
    bcls_pad = jnp.zeros((1, 128), jnp.float32).at[0, :nlbl].set(b_cls)
    wih_t = W_ih.T
    whh_t = W_hh.T
    bih = b_ih.reshape(1, -1)
    bhh = b_hh.reshape(1, -1)

    h1, c1 = pl.pallas_call(
        _lstm_first,
        out_shape=(jax.ShapeDtypeStruct((B, HID), jnp.float32),
                   jax.ShapeDtypeStruct((B, HID), jnp.float32)),
    )(xa, wih_t, whh_t, bih, bhh)

    out = pl.pallas_call(
        _lstm_second,
        out_shape=jax.ShapeDtypeStruct((B, 128), jnp.float32),
    )(xb, h1, c1, wih_t, whh_t, bih, bhh, wcls_pad, bcls_pad)
    return out[:, :nlbl]


# speedup vs baseline: 2.0443x; 1.0095x over previous
import functools

import jax
import jax.numpy as jnp
from jax import lax
from jax.experimental import pallas as pl
from jax.experimental.pallas import tpu as pltpu
from jax.experimental.pallas import tpu_sc as plsc

EMB = 64
HID = 128
B = 1024
T = 50
TH = T // 2
NTOKH = B * TH
CHUNK = 80
LANES = 16


def _make_gather():
    info = plsc.get_sparse_core_info()
    nc, ns = info.num_cores, info.num_subcores
    nw = nc * ns
    tok_w = NTOKH // nw
    nchunk = tok_w // CHUNK

    mesh = plsc.VectorSubcoreMesh(core_axis_name="c", subcore_axis_name="s")

    @functools.partial(
        pl.kernel,
        mesh=mesh,
        compiler_params=pltpu.CompilerParams(needs_layout_passes=False),
        out_type=jax.ShapeDtypeStruct((NTOKH, EMB), jnp.float32),
        scratch_types=[
            pltpu.VMEM((tok_w,), jnp.int32),
            pltpu.VMEM((CHUNK, EMB), jnp.float32),
            pltpu.VMEM((CHUNK, EMB), jnp.float32),
            pltpu.SemaphoreType.DMA,
            pltpu.SemaphoreType.DMA,
        ],
    )
    def gather_k(table_hbm, idx_hbm, out_hbm, idx_v, rows0, rows1,
                 sem0, sem1):
        wid = lax.axis_index("s") * nc + lax.axis_index("c")
        base = wid * tok_w
        pltpu.sync_copy(idx_hbm.at[wid], idx_v)
        bufs = ((rows0, sem0), (rows1, sem1))

        def fire(g, rows_v, sem):
            def fq(q, c):
                iv = idx_v[pl.ds(g * CHUNK + q * LANES, LANES)]
                for jj in range(LANES):
                    pltpu.async_copy(table_hbm.at[iv[jj]],
                                     rows_v.at[q * LANES + jj], sem)
                return c
            lax.fori_loop(0, CHUNK // LANES, fq, 0)

        def drain_out(g, rows_v, sem):
            def dj(j, c):
                pltpu.make_async_copy(table_hbm.at[0], rows_v.at[j],
                                      sem).wait()
                return c
            lax.fori_loop(0, CHUNK, dj, 0)
            pltpu.sync_copy(rows_v,
                            out_hbm.at[pl.ds(base + g * CHUNK, CHUNK)])

        fire(0, *bufs[0])

        def do_pair(k, carry):
            g0 = 2 * k

            @pl.when(g0 + 1 < nchunk)
            def _():
                fire(g0 + 1, *bufs[1])
            drain_out(g0, *bufs[0])

            @pl.when(g0 + 2 < nchunk)
            def _():
                fire(g0 + 2, *bufs[0])

            @pl.when(g0 + 1 < nchunk)
            def _():
                drain_out(g0 + 1, *bufs[1])
            return carry
        lax.fori_loop(0, (nchunk + 1) // 2, do_pair, 0)

    return gather_k


_gather = _make_gather()


def _sigmoid(x):
    return 0.5 * jnp.tanh(0.5 * x) + 0.5


def _lstm_steps(x_ref, wih, whh, b, h, c):
    def step(t, carry):
        h, c = carry
        xt = x_ref[t].astype(jnp.bfloat16)
        gates = jnp.dot(xt, wih, preferred_element_type=jnp.float32)
        gates = gates + jnp.dot(h.astype(jnp.bfloat16), whh,
                                preferred_element_type=jnp.float32)
        gates = gates + b
        i = _sigmoid(gates[:, :HID])
        f = _sigmoid(gates[:, HID:2 * HID])
        g = jnp.tanh(gates[:, 2 * HID:3 * HID])
        o = _sigmoid(gates[:, 3 * HID:])
        c = f * c + i * g
        h = o * jnp.tanh(c)
        return (h, c)
    return lax.fori_loop(0, TH, step, (h, c))


def _lstm_first(x_ref, wih_ref, whh_ref, bih_ref, bhh_ref, h_out, c_out):
    wih = wih_ref[...].astype(jnp.bfloat16)
    whh = whh_ref[...].astype(jnp.bfloat16)
    b = bih_ref[...] + bhh_ref[...]
    h0 = jnp.zeros((B, HID), jnp.float32)
    c0 = jnp.zeros((B, HID), jnp.float32)
    h, c = _lstm_steps(x_ref, wih, whh, b, h0, c0)
    h_out[...] = h
    c_out[...] = c


def _lstm_second(x_ref, h_ref, c_ref, wih_ref, whh_ref, bih_ref, bhh_ref,
                 wcls_ref, bcls_ref, out_ref):
    wih = wih_ref[...].astype(jnp.bfloat16)
    whh = whh_ref[...].astype(jnp.bfloat16)
    b = bih_ref[...] + bhh_ref[...]
    h, c = _lstm_steps(x_ref, wih, whh, b, h_ref[...], c_ref[...])
    out_ref[...] = (jnp.dot(h, wcls_ref[...], preferred_element_type=jnp.float32)
                    + bcls_ref[...])


def kernel(batch_input_ids, emb, W_ih, W_hh, b_ih, b_hh, W_cls, b_cls):
    idx_tb = batch_input_ids.T
    idx_a = idx_tb[:TH].reshape(32, NTOKH // 32)
    idx_b = idx_tb[TH:].reshape(32, NTOKH // 32)
    xa = _gather(emb, idx_a).reshape(TH, B, EMB)
    xb = _gather(emb, idx_b).reshape(TH, B, EMB)

    nlbl = W_cls.shape[0]
    wcls_pad = jnp.zeros((HID, 128), jnp.float32).at[:, :nlbl].set(W_cls.T)
    bcls_pad = jnp.zeros((1, 128), jnp.float32).at[0, :nlbl].set(b_cls)
    wih_t = W_ih.T
    whh_t = W_hh.T
    bih = b_ih.reshape(1, -1)
    bhh = b_hh.reshape(1, -1)

    h1, c1 = pl.pallas_call(
        _lstm_first,
        out_shape=(jax.ShapeDtypeStruct((B, HID), jnp.float32),
                   jax.ShapeDtypeStruct((B, HID), jnp.float32)),
    )(xa, wih_t, whh_t, bih, bhh)

    out = pl.pallas_call(
        _lstm_second,
        out_shape=jax.ShapeDtypeStruct((B, 128), jnp.float32),
    )(xb, h1, c1, wih_t, whh_t, bih, bhh, wcls_pad, bcls_pad)
    return out[:, :nlbl]
